# Initial kernel scaffold; baseline (speedup 1.0000x reference)
import jax
import jax.numpy as jnp
from jax.experimental import pallas as pl


def kernel(x, edge_index, batch, W_enc, b_enc, W_cls, b_cls):
    raise NotImplementedError("write your pallas kernel here")


# speedup vs baseline: 3.6944x; 3.6944x over previous
import functools

import jax
import jax.numpy as jnp
from jax import lax
from jax.experimental import pallas as pl
from jax.experimental.pallas import tpu as pltpu
from jax.experimental.pallas import tpu_sc as plsc

N_NODES = 10000
D = 128
NSEG = 512
CNT_W = 128

NC = 2
NS = 16
CHUNK = 40
NCHUNK = N_NODES // CHUNK
NWORK = NC * NS
NITER = -(-NCHUNK // NWORK)


@functools.partial(
    pl.kernel,
    out_type=[
        jax.ShapeDtypeStruct((NC, NSEG, D), jnp.float32),
        jax.ShapeDtypeStruct((NC, NSEG, CNT_W), jnp.float32),
    ],
    mesh=plsc.VectorSubcoreMesh(core_axis_name="c", subcore_axis_name="s"),
    scratch_types=[
        pltpu.VMEM((CHUNK, D), jnp.float32),
        pltpu.VMEM((CHUNK,), jnp.int32),
        pltpu.VMEM((CHUNK, CNT_W), jnp.float32),
        pltpu.VMEM((NSEG // NS, D), jnp.float32),
        pltpu.VMEM_SHARED((NSEG, D), jnp.float32),
        pltpu.VMEM_SHARED((NSEG, CNT_W), jnp.float32),
    ],
)
def _sc_segment_sum(x_hbm, batch_hbm, acc_out, cnt_out,
                    rows_v, idx_v, ones_v, z_d_v, acc_sh, cnt_sh):
    cid = lax.axis_index("c")
    sid = lax.axis_index("s")
    wid = sid * NC + cid

    zvec = jnp.zeros((16,), jnp.float32)
    onevec = jnp.ones((16,), jnp.float32)
    for i in range(NSEG // NS):
        for j in range(D // 16):
            z_d_v[i, pl.ds(j * 16, 16)] = zvec
    for i in range(CHUNK):
        for j in range(CNT_W // 16):
            ones_v[i, pl.ds(j * 16, 16)] = onevec

    rows_per_tile = NSEG // NS
    pltpu.sync_copy(z_d_v, acc_sh.at[pl.ds(sid * rows_per_tile, rows_per_tile)])
    pltpu.sync_copy(z_d_v, cnt_sh.at[pl.ds(sid * rows_per_tile, rows_per_tile)])
    plsc.subcore_barrier()

    for it in range(NITER):
        chunk = it * NWORK + wid

        def process(chunk=chunk):
            pltpu.sync_copy(batch_hbm.at[pl.ds(chunk * CHUNK, CHUNK)], idx_v)
            pltpu.sync_copy(x_hbm.at[pl.ds(chunk * CHUNK, CHUNK)], rows_v)
            pltpu.sync_copy(rows_v, acc_sh.at[idx_v], add=True)
            pltpu.sync_copy(ones_v, cnt_sh.at[idx_v], add=True)

        if (it + 1) * NWORK <= NCHUNK:
            process()
        else:
            pl.when(chunk < NCHUNK)(process)

    plsc.subcore_barrier()

    @pl.when(sid == 0)
    def _write_out():
        pltpu.sync_copy(acc_sh, acc_out.at[cid])
        pltpu.sync_copy(cnt_sh, cnt_out.at[cid])


def _finish_body(acc_ref, cnt_ref, w1_ref, b1_ref, w2_ref, b2_ref, out_ref):
    a = acc_ref[...]
    sums = a[0] + a[1]
    c = cnt_ref[...]
    counts = c[0, :, 0:1] + c[1, :, 0:1]
    pooled = sums / jnp.maximum(counts, 1.0)
    mask = (counts > 0.0).astype(jnp.float32)
    h = (jnp.dot(pooled, w1_ref[...], preferred_element_type=jnp.float32,
                 precision=lax.Precision.HIGHEST)
         + b1_ref[...] * mask)
    out_ref[...] = (jnp.dot(h, w2_ref[...], preferred_element_type=jnp.float32,
                            precision=lax.Precision.HIGHEST)
                    + b2_ref[...])


_finish = pl.pallas_call(
    _finish_body,
    out_shape=jax.ShapeDtypeStruct((NSEG, D), jnp.float32),
)


def kernel(x, edge_index, batch, W_enc, b_enc, W_cls, b_cls):
    acc, cnt = _sc_segment_sum(x, batch)
    return _finish(acc, cnt, W_enc, b_enc.reshape(1, D), W_cls,
                   b_cls.reshape(1, D))


# speedup vs baseline: 4.4333x; 1.2000x over previous
import functools

import jax
import jax.numpy as jnp
from jax import lax
from jax.experimental import pallas as pl
from jax.experimental.pallas import tpu as pltpu
from jax.experimental.pallas import tpu_sc as plsc

N_NODES = 10000
D = 128
NSEG = 512
CNT_W = 128

NC = 2
NS = 16
CHUNK = 40
NCHUNK = N_NODES // CHUNK
NWORK = NC * NS
NITER = -(-NCHUNK // NWORK)


@functools.partial(
    pl.kernel,
    out_type=[
        jax.ShapeDtypeStruct((NC, NSEG, D), jnp.float32),
        jax.ShapeDtypeStruct((NC, NSEG, CNT_W), jnp.float32),
    ],
    mesh=plsc.VectorSubcoreMesh(core_axis_name="c", subcore_axis_name="s"),
    scratch_types=[
        pltpu.VMEM((2, CHUNK, D), jnp.float32),
        pltpu.VMEM((2, CHUNK), jnp.int32),
        pltpu.VMEM((CHUNK, CNT_W), jnp.float32),
        pltpu.VMEM((NSEG // NS, D), jnp.float32),
        pltpu.VMEM_SHARED((NSEG, D), jnp.float32),
        pltpu.VMEM_SHARED((NSEG, CNT_W), jnp.float32),
        pltpu.SemaphoreType.DMA,
        pltpu.SemaphoreType.DMA,
        pltpu.SemaphoreType.DMA,
        pltpu.SemaphoreType.DMA,
        pltpu.SemaphoreType.DMA,
        pltpu.SemaphoreType.DMA,
    ],
)
def _sc_segment_sum(x_hbm, batch_hbm, acc_out, cnt_out,
                    rows_v, idx_v, ones_v, z_d_v, acc_sh, cnt_sh,
                    sfi0, sfi1, sfr0, sfr1, ssc0, ssc1):
    cid = lax.axis_index("c")
    sid = lax.axis_index("s")
    wid = sid * NC + cid

    zvec = jnp.zeros((16,), jnp.float32)
    onevec = jnp.ones((16,), jnp.float32)
    for i in range(NSEG // NS):
        for j in range(D // 16):
            z_d_v[i, pl.ds(j * 16, 16)] = zvec
    for i in range(CHUNK):
        for j in range(CNT_W // 16):
            ones_v[i, pl.ds(j * 16, 16)] = onevec

    rows_per_tile = NSEG // NS
    pltpu.sync_copy(z_d_v, acc_sh.at[pl.ds(sid * rows_per_tile, rows_per_tile)])
    pltpu.sync_copy(z_d_v, cnt_sh.at[pl.ds(sid * rows_per_tile, rows_per_tile)])
    plsc.subcore_barrier()

    sfi = [sfi0, sfi1]
    sfr = [sfr0, sfr1]
    ssc = [ssc0, ssc1]
    NPIPE = NITER - 1
    fetch_h = {}
    scat_h = {}

    def start_fetch(it):
        b = it % 2
        chunk = it * NWORK + wid
        hi = pltpu.async_copy(batch_hbm.at[pl.ds(chunk * CHUNK, CHUNK)],
                              idx_v.at[b], sfi[b])
        hr = pltpu.async_copy(x_hbm.at[pl.ds(chunk * CHUNK, CHUNK)],
                              rows_v.at[b], sfr[b])
        fetch_h[it] = (hi, hr)

    start_fetch(0)
    for it in range(NPIPE):
        b = it % 2
        if it + 1 < NPIPE:
            if it >= 1:
                for h in scat_h.pop(it - 1):
                    h.wait()
            start_fetch(it + 1)
        for h in fetch_h.pop(it):
            h.wait()
        hs = pltpu.async_copy(rows_v.at[b], acc_sh.at[idx_v.at[b]], ssc[b],
                              add=True)
        hc = pltpu.async_copy(ones_v, cnt_sh.at[idx_v.at[b]], ssc[b],
                              add=True)
        scat_h[it] = (hs, hc)
    for hs in scat_h.values():
        for h in hs:
            h.wait()

    last_chunk = NPIPE * NWORK + wid

    @pl.when(last_chunk < NCHUNK)
    def _tail():
        pltpu.sync_copy(batch_hbm.at[pl.ds(last_chunk * CHUNK, CHUNK)],
                        idx_v.at[0])
        pltpu.sync_copy(x_hbm.at[pl.ds(last_chunk * CHUNK, CHUNK)],
                        rows_v.at[0])
        pltpu.sync_copy(rows_v.at[0], acc_sh.at[idx_v.at[0]], add=True)
        pltpu.sync_copy(ones_v, cnt_sh.at[idx_v.at[0]], add=True)

    plsc.subcore_barrier()

    @pl.when(sid == 0)
    def _write_out():
        pltpu.sync_copy(acc_sh, acc_out.at[cid])
        pltpu.sync_copy(cnt_sh, cnt_out.at[cid])


def _finish_body(acc_ref, cnt_ref, w1_ref, b1_ref, w2_ref, b2_ref, out_ref):
    a = acc_ref[...]
    sums = a[0] + a[1]
    c = cnt_ref[...]
    counts = c[0, :, 0:1] + c[1, :, 0:1]
    pooled = sums / jnp.maximum(counts, 1.0)
    mask = (counts > 0.0).astype(jnp.float32)
    h = (jnp.dot(pooled, w1_ref[...], preferred_element_type=jnp.float32,
                 precision=lax.Precision.HIGHEST)
         + b1_ref[...] * mask)
    out_ref[...] = (jnp.dot(h, w2_ref[...], preferred_element_type=jnp.float32,
                            precision=lax.Precision.HIGHEST)
                    + b2_ref[...])


_finish = pl.pallas_call(
    _finish_body,
    out_shape=jax.ShapeDtypeStruct((NSEG, D), jnp.float32),
)


def kernel(x, edge_index, batch, W_enc, b_enc, W_cls, b_cls):
    acc, cnt = _sc_segment_sum(x, batch)
    return _finish(acc, cnt, W_enc, b_enc.reshape(1, D), W_cls,
                   b_cls.reshape(1, D))


# speedup vs baseline: 4.4890x; 1.0126x over previous
import functools

import jax
import jax.numpy as jnp
from jax import lax
from jax.experimental import pallas as pl
from jax.experimental.pallas import tpu as pltpu
from jax.experimental.pallas import tpu_sc as plsc

N_NODES = 10000
D = 128
NSEG = 512

NC = 2
NS = 16
CHUNK = 80
NCHUNK = N_NODES // CHUNK
NWORK = NC * NS
NITER = -(-NCHUNK // NWORK)


@functools.partial(
    pl.kernel,
    out_type=[
        jax.ShapeDtypeStruct((NC, NSEG, D), jnp.float32),
        jax.ShapeDtypeStruct((NC, NSEG, D), jnp.float32),
    ],
    mesh=plsc.VectorSubcoreMesh(core_axis_name="c", subcore_axis_name="s"),
    scratch_types=[
        pltpu.VMEM((2, CHUNK, D), jnp.float32),
        pltpu.VMEM((2, CHUNK), jnp.int32),
        pltpu.VMEM((CHUNK, D), jnp.float32),
        pltpu.VMEM((NSEG // NS, D), jnp.float32),
        pltpu.VMEM_SHARED((NSEG, D), jnp.float32),
        pltpu.VMEM_SHARED((NSEG, D), jnp.float32),
        pltpu.SemaphoreType.DMA,
        pltpu.SemaphoreType.DMA,
        pltpu.SemaphoreType.DMA,
        pltpu.SemaphoreType.DMA,
        pltpu.SemaphoreType.DMA,
        pltpu.SemaphoreType.DMA,
    ],
)
def _sc_segment_sum(x_hbm, batch_hbm, acc_out, cnt_out,
                    rows_v, idx_v, ones_v, z_d_v, acc_sh, cnt_sh,
                    sfi0, sfi1, sfr0, sfr1, ssc0, ssc1):
    cid = lax.axis_index("c")
    sid = lax.axis_index("s")
    wid = sid * NC + cid

    zvec = jnp.zeros((16,), jnp.float32)
    onevec = jnp.ones((16,), jnp.float32)
    for i in range(NSEG // NS):
        for j in range(D // 16):
            z_d_v[i, pl.ds(j * 16, 16)] = zvec
    for i in range(CHUNK):
        for j in range(D // 16):
            ones_v[i, pl.ds(j * 16, 16)] = onevec

    rows_per_tile = NSEG // NS
    pltpu.sync_copy(z_d_v, acc_sh.at[pl.ds(sid * rows_per_tile, rows_per_tile)])
    pltpu.sync_copy(z_d_v, cnt_sh.at[pl.ds(sid * rows_per_tile, rows_per_tile)])
    plsc.subcore_barrier()

    sfi = [sfi0, sfi1]
    sfr = [sfr0, sfr1]
    ssc = [ssc0, ssc1]
    NPIPE = NITER - 1
    fetch_h = {}
    scat_h = {}

    def start_fetch(it):
        b = it % 2
        chunk = it * NWORK + wid
        hi = pltpu.async_copy(batch_hbm.at[pl.ds(chunk * CHUNK, CHUNK)],
                              idx_v.at[b], sfi[b])
        hr = pltpu.async_copy(x_hbm.at[pl.ds(chunk * CHUNK, CHUNK)],
                              rows_v.at[b], sfr[b])
        fetch_h[it] = (hi, hr)

    start_fetch(0)
    for it in range(NPIPE):
        b = it % 2
        if it + 1 < NPIPE:
            if it >= 1:
                for h in scat_h.pop(it - 1):
                    h.wait()
            start_fetch(it + 1)
        for h in fetch_h.pop(it):
            h.wait()
        hs = pltpu.async_copy(rows_v.at[b], acc_sh.at[idx_v.at[b]], ssc[b],
                              add=True)
        hc = pltpu.async_copy(ones_v, cnt_sh.at[idx_v.at[b]], ssc[b],
                              add=True)
        scat_h[it] = (hs, hc)
    for hs in scat_h.values():
        for h in hs:
            h.wait()

    last_chunk = NPIPE * NWORK + wid

    @pl.when(last_chunk < NCHUNK)
    def _tail():
        pltpu.sync_copy(batch_hbm.at[pl.ds(last_chunk * CHUNK, CHUNK)],
                        idx_v.at[0])
        pltpu.sync_copy(x_hbm.at[pl.ds(last_chunk * CHUNK, CHUNK)],
                        rows_v.at[0])
        pltpu.sync_copy(rows_v.at[0], acc_sh.at[idx_v.at[0]], add=True)
        pltpu.sync_copy(ones_v, cnt_sh.at[idx_v.at[0]], add=True)

    plsc.subcore_barrier()

    @pl.when(sid == 0)
    def _write_out():
        pltpu.sync_copy(acc_sh, acc_out.at[cid])
        pltpu.sync_copy(cnt_sh, cnt_out.at[cid])


def _finish_body(acc_ref, cnt_ref, w1_ref, b1_ref, w2_ref, b2_ref, out_ref):
    a = acc_ref[...]
    sums = a[0] + a[1]
    c = cnt_ref[...]
    counts = c[0, :, 0:1] + c[1, :, 0:1]
    pooled = sums / jnp.maximum(counts, 1.0)
    mask = (counts > 0.0).astype(jnp.float32)
    h = jnp.dot(pooled, w1_ref[...], preferred_element_type=jnp.float32) \
        + b1_ref[...] * mask
    out_ref[...] = jnp.dot(h, w2_ref[...], preferred_element_type=jnp.float32) \
        + b2_ref[...]


_finish = pl.pallas_call(
    _finish_body,
    out_shape=jax.ShapeDtypeStruct((NSEG, D), jnp.float32),
)


def kernel(x, edge_index, batch, W_enc, b_enc, W_cls, b_cls):
    acc, cnt = _sc_segment_sum(x, batch)
    return _finish(acc, cnt, W_enc, b_enc.reshape(1, D), W_cls,
                   b_cls.reshape(1, D))


# speedup vs baseline: 4.7194x; 1.0513x over previous
import functools

import jax
import jax.numpy as jnp
from jax import lax
from jax.experimental import pallas as pl
from jax.experimental.pallas import tpu as pltpu
from jax.experimental.pallas import tpu_sc as plsc

N_NODES = 10000
D = 128
NSEG = 512

NC = 2
NS = 16
CHUNK = 80
NCHUNK = N_NODES // CHUNK
NWORK = NC * NS
NITER = -(-NCHUNK // NWORK)


@functools.partial(
    pl.kernel,
    out_type=[
        jax.ShapeDtypeStruct((NC, NSEG, D), jnp.float32),
        jax.ShapeDtypeStruct((NC, NSEG, D), jnp.float32),
    ],
    mesh=plsc.VectorSubcoreMesh(core_axis_name="c", subcore_axis_name="s"),
    scratch_types=[
        pltpu.VMEM((2, CHUNK, D), jnp.float32),
        pltpu.VMEM((NITER, CHUNK), jnp.int32),
        pltpu.VMEM((CHUNK, D), jnp.float32),
        pltpu.VMEM((NSEG // NS, D), jnp.float32),
        pltpu.VMEM_SHARED((NSEG, D), jnp.float32),
        pltpu.VMEM_SHARED((NSEG, D), jnp.float32),
        pltpu.SemaphoreType.DMA,
        pltpu.SemaphoreType.DMA,
        pltpu.SemaphoreType.DMA,
        pltpu.SemaphoreType.DMA,
        pltpu.SemaphoreType.DMA,
        pltpu.SemaphoreType.DMA,
    ],
)
def _sc_segment_sum(x_hbm, batch_hbm, acc_out, cnt_out,
                    rows_v, idx_v, ones_v, z_d_v, acc_sh, cnt_sh,
                    sfi0, sfi1, sfr0, sfr1, ssc0, ssc1):
    cid = lax.axis_index("c")
    sid = lax.axis_index("s")
    wid = sid * NC + cid

    zvec = jnp.zeros((16,), jnp.float32)
    onevec = jnp.ones((16,), jnp.float32)
    for i in range(NSEG // NS):
        for j in range(D // 16):
            z_d_v[i, pl.ds(j * 16, 16)] = zvec
    for i in range(CHUNK):
        for j in range(D // 16):
            ones_v[i, pl.ds(j * 16, 16)] = onevec

    rows_per_tile = NSEG // NS
    pltpu.sync_copy(z_d_v, acc_sh.at[pl.ds(sid * rows_per_tile, rows_per_tile)])
    pltpu.sync_copy(z_d_v, cnt_sh.at[pl.ds(sid * rows_per_tile, rows_per_tile)])
    plsc.subcore_barrier()

    sfr = [sfr0, sfr1]
    ssc = [ssc0, ssc1]
    tail_on = NITER * NWORK > NCHUNK
    fetch_h = {}
    scat_h = {}

    def chunk_of(it):
        return it * NWORK + wid

    def start_rows_fetch(it):
        b = it % 2
        fetch_h[it] = pltpu.async_copy(
            x_hbm.at[pl.ds(chunk_of(it) * CHUNK, CHUNK)], rows_v.at[b], sfr[b])

    def start_scatters(it):
        b = it % 2
        hs = pltpu.async_copy(rows_v.at[b], acc_sh.at[idx_v.at[it]], ssc[b],
                              add=True)
        hc = pltpu.async_copy(ones_v, cnt_sh.at[idx_v.at[it]], ssc[b],
                              add=True)
        scat_h[it] = (hs, hc)

    idx_h = []
    for it in range(NITER - 1):
        idx_h.append(pltpu.async_copy(
            batch_hbm.at[pl.ds(chunk_of(it) * CHUNK, CHUNK)],
            idx_v.at[it], sfi0))
    last = NITER - 1
    if tail_on:
        @pl.when(chunk_of(last) < NCHUNK)
        def _fetch_last_idx():
            pltpu.async_copy(batch_hbm.at[pl.ds(chunk_of(last) * CHUNK, CHUNK)],
                             idx_v.at[last], sfi1)
    else:
        idx_h.append(pltpu.async_copy(
            batch_hbm.at[pl.ds(chunk_of(last) * CHUNK, CHUNK)],
            idx_v.at[last], sfi1))

    start_rows_fetch(0)
    for it in range(NITER - 1):
        nxt = it + 1
        if nxt < NITER:
            if it >= 1:
                for h in scat_h.pop(it - 1):
                    h.wait()
            if nxt == last and tail_on:
                @pl.when(chunk_of(last) < NCHUNK)
                def _fetch_last_rows():
                    start_rows_fetch(last)
                    fetch_h.pop(last)
            else:
                start_rows_fetch(nxt)
        fetch_h.pop(it).wait()
        if it == 0:
            for h in idx_h:
                h.wait()
        start_scatters(it)

    if tail_on:
        @pl.when(chunk_of(last) < NCHUNK)
        def _tail_work():
            b = last % 2
            pltpu.make_async_copy(
                batch_hbm.at[pl.ds(chunk_of(last) * CHUNK, CHUNK)],
                idx_v.at[last], sfi1).wait()
            pltpu.make_async_copy(
                x_hbm.at[pl.ds(chunk_of(last) * CHUNK, CHUNK)],
                rows_v.at[b], sfr[b]).wait()
            pltpu.sync_copy(rows_v.at[b], acc_sh.at[idx_v.at[last]], add=True)
            pltpu.sync_copy(ones_v, cnt_sh.at[idx_v.at[last]], add=True)

    for hs in scat_h.values():
        for h in hs:
            h.wait()

    plsc.subcore_barrier()

    @pl.when(sid == 0)
    def _write_acc():
        pltpu.sync_copy(acc_sh, acc_out.at[cid])

    @pl.when(sid == 1)
    def _write_cnt():
        pltpu.sync_copy(cnt_sh, cnt_out.at[cid])


def _finish_body(acc_ref, cnt_ref, w1_ref, b1_ref, w2_ref, b2_ref, out_ref):
    a = acc_ref[...]
    sums = a[0] + a[1]
    c = cnt_ref[...]
    counts = c[0, :, 0:1] + c[1, :, 0:1]
    pooled = sums / jnp.maximum(counts, 1.0)
    mask = (counts > 0.0).astype(jnp.float32)
    h = jnp.dot(pooled, w1_ref[...], preferred_element_type=jnp.float32) \
        + b1_ref[...] * mask
    out_ref[...] = jnp.dot(h, w2_ref[...], preferred_element_type=jnp.float32) \
        + b2_ref[...]


_finish = pl.pallas_call(
    _finish_body,
    out_shape=jax.ShapeDtypeStruct((NSEG, D), jnp.float32),
)


def kernel(x, edge_index, batch, W_enc, b_enc, W_cls, b_cls):
    acc, cnt = _sc_segment_sum(x, batch)
    return _finish(acc, cnt, W_enc, b_enc.reshape(1, D), W_cls,
                   b_cls.reshape(1, D))


# speedup vs baseline: 4.7462x; 1.0057x over previous
import functools

import jax
import jax.numpy as jnp
from jax import lax
from jax.experimental import pallas as pl
from jax.experimental.pallas import tpu as pltpu
from jax.experimental.pallas import tpu_sc as plsc

N_NODES = 10000
D = 128
NSEG = 512

NC = 2
NS = 16
CHUNK = 80
NCHUNK = N_NODES // CHUNK
NWORK = NC * NS
NITER = -(-NCHUNK // NWORK)


@functools.partial(
    pl.kernel,
    out_type=[
        jax.ShapeDtypeStruct((NC, NSEG, D), jnp.float32),
        jax.ShapeDtypeStruct((NC, NSEG, D), jnp.float32),
    ],
    mesh=plsc.VectorSubcoreMesh(core_axis_name="c", subcore_axis_name="s"),
    scratch_types=[
        pltpu.VMEM((2, CHUNK, D), jnp.float32),
        pltpu.VMEM((NITER, CHUNK), jnp.int32),
        pltpu.VMEM((CHUNK, D), jnp.float32),
        pltpu.VMEM((NSEG // NS, D), jnp.float32),
        pltpu.VMEM_SHARED((NSEG, D), jnp.float32),
        pltpu.VMEM_SHARED((NSEG, D), jnp.float32),
        pltpu.SemaphoreType.DMA,
        pltpu.SemaphoreType.DMA,
        pltpu.SemaphoreType.DMA,
        pltpu.SemaphoreType.DMA,
        pltpu.SemaphoreType.DMA,
        pltpu.SemaphoreType.DMA,
        pltpu.SemaphoreType.DMA,
    ],
)
def _sc_segment_sum(x_hbm, batch_hbm, acc_out, cnt_out,
                    rows_v, idx_v, ones_v, z_d_v, acc_sh, cnt_sh,
                    sfi0, sfi1, sfr0, sfr1, ssc0, ssc1, scnt):
    cid = lax.axis_index("c")
    sid = lax.axis_index("s")
    wid = sid * NC + cid

    zvec = jnp.zeros((16,), jnp.float32)
    onevec = jnp.ones((16,), jnp.float32)
    for i in range(NSEG // NS):
        for j in range(D // 16):
            z_d_v[i, pl.ds(j * 16, 16)] = zvec
    for i in range(CHUNK):
        for j in range(D // 16):
            ones_v[i, pl.ds(j * 16, 16)] = onevec

    rows_per_tile = NSEG // NS
    pltpu.sync_copy(z_d_v, acc_sh.at[pl.ds(sid * rows_per_tile, rows_per_tile)])
    pltpu.sync_copy(z_d_v, cnt_sh.at[pl.ds(sid * rows_per_tile, rows_per_tile)])
    plsc.subcore_barrier()

    sfr = [sfr0, sfr1]
    ssc = [ssc0, ssc1]
    tail_on = NITER * NWORK > NCHUNK
    fetch_h = {}
    scat_h = {}

    def chunk_of(it):
        return it * NWORK + wid

    def start_rows_fetch(it):
        b = it % 2
        fetch_h[it] = pltpu.async_copy(
            x_hbm.at[pl.ds(chunk_of(it) * CHUNK, CHUNK)], rows_v.at[b], sfr[b])

    def start_scatters(it):
        b = it % 2
        scat_h[it] = pltpu.async_copy(rows_v.at[b], acc_sh.at[idx_v.at[it]],
                                      ssc[b], add=True)

    idx_h = []
    for it in range(NITER - 1):
        idx_h.append(pltpu.async_copy(
            batch_hbm.at[pl.ds(chunk_of(it) * CHUNK, CHUNK)],
            idx_v.at[it], sfi0))
    last = NITER - 1
    if tail_on:
        @pl.when(chunk_of(last) < NCHUNK)
        def _fetch_last_idx():
            pltpu.async_copy(batch_hbm.at[pl.ds(chunk_of(last) * CHUNK, CHUNK)],
                             idx_v.at[last], sfi1)
    else:
        idx_h.append(pltpu.async_copy(
            batch_hbm.at[pl.ds(chunk_of(last) * CHUNK, CHUNK)],
            idx_v.at[last], sfi1))

    start_rows_fetch(0)
    for it in range(NITER - 1):
        nxt = it + 1
        if nxt < NITER:
            if it >= 1:
                scat_h.pop(it - 1).wait()
            if nxt == last and tail_on:
                @pl.when(chunk_of(last) < NCHUNK)
                def _fetch_last_rows():
                    start_rows_fetch(last)
                    fetch_h.pop(last)
            else:
                start_rows_fetch(nxt)
        fetch_h.pop(it).wait()
        if it == 0:
            for h in idx_h:
                h.wait()
            cnt_h = []
            for jt in range(NITER - 1):
                cnt_h.append(pltpu.async_copy(
                    ones_v, cnt_sh.at[idx_v.at[jt]], scnt, add=True))
        start_scatters(it)

    if tail_on:
        @pl.when(chunk_of(last) < NCHUNK)
        def _tail_work():
            b = last % 2
            pltpu.make_async_copy(
                batch_hbm.at[pl.ds(chunk_of(last) * CHUNK, CHUNK)],
                idx_v.at[last], sfi1).wait()
            pltpu.async_copy(ones_v, cnt_sh.at[idx_v.at[last]], scnt,
                             add=True)
            pltpu.make_async_copy(
                x_hbm.at[pl.ds(chunk_of(last) * CHUNK, CHUNK)],
                rows_v.at[b], sfr[b]).wait()
            pltpu.sync_copy(rows_v.at[b], acc_sh.at[idx_v.at[last]], add=True)
            pltpu.make_async_copy(ones_v, cnt_sh.at[idx_v.at[last]],
                                  scnt).wait()

    for h in scat_h.values():
        h.wait()
    for h in cnt_h:
        h.wait()

    plsc.subcore_barrier()

    @pl.when(sid == 0)
    def _write_acc():
        pltpu.sync_copy(acc_sh, acc_out.at[cid])

    @pl.when(sid == 1)
    def _write_cnt():
        pltpu.sync_copy(cnt_sh, cnt_out.at[cid])


def _finish_body(acc_ref, cnt_ref, w1_ref, b1_ref, w2_ref, b2_ref, out_ref):
    a = acc_ref[...]
    sums = a[0] + a[1]
    c = cnt_ref[...]
    counts = c[0] + c[1]
    pooled = sums / jnp.maximum(counts, 1.0)
    mask = (counts > 0.0).astype(jnp.float32)
    h = jnp.dot(pooled, w1_ref[...], preferred_element_type=jnp.float32) \
        + b1_ref[...] * mask
    out_ref[...] = jnp.dot(h, w2_ref[...], preferred_element_type=jnp.float32) \
        + b2_ref[...]


_finish = pl.pallas_call(
    _finish_body,
    out_shape=jax.ShapeDtypeStruct((NSEG, D), jnp.float32),
)


def kernel(x, edge_index, batch, W_enc, b_enc, W_cls, b_cls):
    acc, cnt = _sc_segment_sum(x, batch)
    return _finish(acc, cnt[:, :, 0:1], W_enc, b_enc.reshape(1, D), W_cls,
                   b_cls.reshape(1, D))


# speedup vs baseline: 4.7512x; 1.0011x over previous
import functools

import jax
import jax.numpy as jnp
from jax import lax
from jax.experimental import pallas as pl
from jax.experimental.pallas import tpu as pltpu
from jax.experimental.pallas import tpu_sc as plsc

N_NODES = 10000
D = 128
NSEG = 512

NC = 2
NS = 16
CHUNK = 80
NCHUNK = N_NODES // CHUNK
NWORK = NC * NS
NITER = -(-NCHUNK // NWORK)


@functools.partial(
    pl.kernel,
    out_type=[
        jax.ShapeDtypeStruct((NC, NSEG, D), jnp.float32),
        jax.ShapeDtypeStruct((NC, NSEG, D), jnp.float32),
    ],
    mesh=plsc.VectorSubcoreMesh(core_axis_name="c", subcore_axis_name="s"),
    scratch_types=[
        pltpu.VMEM((3, CHUNK, D), jnp.float32),
        pltpu.VMEM((NITER, CHUNK), jnp.int32),
        pltpu.VMEM((CHUNK, D), jnp.float32),
        pltpu.VMEM((NSEG // NS, D), jnp.float32),
        pltpu.VMEM_SHARED((NSEG, D), jnp.float32),
        pltpu.VMEM_SHARED((NSEG, D), jnp.float32),
        pltpu.SemaphoreType.DMA,
        pltpu.SemaphoreType.DMA,
        pltpu.SemaphoreType.DMA,
        pltpu.SemaphoreType.DMA,
        pltpu.SemaphoreType.DMA,
        pltpu.SemaphoreType.DMA,
        pltpu.SemaphoreType.DMA,
        pltpu.SemaphoreType.DMA,
        pltpu.SemaphoreType.DMA,
    ],
)
def _sc_segment_sum(x_hbm, batch_hbm, acc_out, cnt_out,
                    rows_v, idx_v, ones_v, z_d_v, acc_sh, cnt_sh,
                    sfi0, sfi1, sfr0, sfr1, sfr2, ssc0, ssc1, ssc2, scnt):
    cid = lax.axis_index("c")
    sid = lax.axis_index("s")
    wid = sid * NC + cid

    zvec = jnp.zeros((16,), jnp.float32)
    onevec = jnp.ones((16,), jnp.float32)
    for i in range(NSEG // NS):
        for j in range(D // 16):
            z_d_v[i, pl.ds(j * 16, 16)] = zvec
    for i in range(CHUNK):
        for j in range(D // 16):
            ones_v[i, pl.ds(j * 16, 16)] = onevec

    rows_per_tile = NSEG // NS
    pltpu.sync_copy(z_d_v, acc_sh.at[pl.ds(sid * rows_per_tile, rows_per_tile)])
    pltpu.sync_copy(z_d_v, cnt_sh.at[pl.ds(sid * rows_per_tile, rows_per_tile)])
    plsc.subcore_barrier()

    sfr = [sfr0, sfr1, sfr2]
    ssc = [ssc0, ssc1, ssc2]
    NBUF = 3
    tail_on = NITER * NWORK > NCHUNK
    fetch_h = {}
    scat_h = {}

    def chunk_of(it):
        return it * NWORK + wid

    def start_rows_fetch(it):
        b = it % NBUF
        fetch_h[it] = pltpu.async_copy(
            x_hbm.at[pl.ds(chunk_of(it) * CHUNK, CHUNK)], rows_v.at[b], sfr[b])

    def start_scatters(it):
        b = it % NBUF
        scat_h[it] = pltpu.async_copy(rows_v.at[b], acc_sh.at[idx_v.at[it]],
                                      ssc[b], add=True)

    idx_h = []
    for it in range(NITER - 1):
        idx_h.append(pltpu.async_copy(
            batch_hbm.at[pl.ds(chunk_of(it) * CHUNK, CHUNK)],
            idx_v.at[it], sfi0))
    last = NITER - 1
    if tail_on:
        @pl.when(chunk_of(last) < NCHUNK)
        def _fetch_last_idx():
            pltpu.async_copy(batch_hbm.at[pl.ds(chunk_of(last) * CHUNK, CHUNK)],
                             idx_v.at[last], sfi1)
    else:
        idx_h.append(pltpu.async_copy(
            batch_hbm.at[pl.ds(chunk_of(last) * CHUNK, CHUNK)],
            idx_v.at[last], sfi1))

    start_rows_fetch(0)
    for it in range(NITER - 1):
        nxt = it + 1
        if nxt < NITER:
            if nxt >= NBUF:
                scat_h.pop(nxt - NBUF).wait()
            if nxt == last and tail_on:
                @pl.when(chunk_of(last) < NCHUNK)
                def _fetch_last_rows():
                    start_rows_fetch(last)
                    fetch_h.pop(last)
            else:
                start_rows_fetch(nxt)
        fetch_h.pop(it).wait()
        if it == 0:
            for h in idx_h:
                h.wait()
            cnt_h = []
            for jt in range(NITER - 1):
                cnt_h.append(pltpu.async_copy(
                    ones_v, cnt_sh.at[idx_v.at[jt]], scnt, add=True))
        start_scatters(it)

    if tail_on:
        @pl.when(chunk_of(last) < NCHUNK)
        def _tail_work():
            b = last % NBUF
            pltpu.make_async_copy(
                batch_hbm.at[pl.ds(chunk_of(last) * CHUNK, CHUNK)],
                idx_v.at[last], sfi1).wait()
            pltpu.async_copy(ones_v, cnt_sh.at[idx_v.at[last]], scnt,
                             add=True)
            pltpu.make_async_copy(
                x_hbm.at[pl.ds(chunk_of(last) * CHUNK, CHUNK)],
                rows_v.at[b], sfr[b]).wait()
            pltpu.sync_copy(rows_v.at[b], acc_sh.at[idx_v.at[last]], add=True)
            pltpu.make_async_copy(ones_v, cnt_sh.at[idx_v.at[last]],
                                  scnt).wait()

    for h in scat_h.values():
        h.wait()
    for h in cnt_h:
        h.wait()

    plsc.subcore_barrier()

    @pl.when(sid == 0)
    def _write_acc():
        pltpu.sync_copy(acc_sh, acc_out.at[cid])

    @pl.when(sid == 1)
    def _write_cnt():
        pltpu.sync_copy(cnt_sh, cnt_out.at[cid])


def _finish_body(acc_ref, cnt_ref, w1_ref, b1_ref, w2_ref, b2_ref, out_ref):
    a = acc_ref[...]
    sums = a[0] + a[1]
    c = cnt_ref[...]
    counts = c[0] + c[1]
    pooled = sums / jnp.maximum(counts, 1.0)
    mask = (counts > 0.0).astype(jnp.float32)
    h = jnp.dot(pooled, w1_ref[...], preferred_element_type=jnp.float32) \
        + b1_ref[...] * mask
    out_ref[...] = jnp.dot(h, w2_ref[...], preferred_element_type=jnp.float32) \
        + b2_ref[...]


_finish = pl.pallas_call(
    _finish_body,
    out_shape=jax.ShapeDtypeStruct((NSEG, D), jnp.float32),
)


def kernel(x, edge_index, batch, W_enc, b_enc, W_cls, b_cls):
    acc, cnt = _sc_segment_sum(x, batch)
    return _finish(acc, cnt[:, :, 0:1], W_enc, b_enc.reshape(1, D), W_cls,
                   b_cls.reshape(1, D))
